# baseline (speedup 1.0000x reference)
import functools
import jax
import jax.numpy as jnp
from jax import lax
from jax.experimental import pallas as pl
from jax.experimental.pallas import tpu as pltpu
from jax.experimental.pallas import tpu_sc as plsc

B = 16384
D = 64
NW = 32
BPW = B // NW
CB = 128
NCHUNK = BPW // CB
L = 16


def _tower_kernel(user_table, item_table, uidx_hbm, iidx_hbm, out_hbm,
                  uidx_v, iidx_v, rows_u, rows_v, pbuf, out_v, sem):
  wid = lax.axis_index("s") * 2 + lax.axis_index("c")
  base = wid * BPW

  for c in range(NCHUNK):
    pltpu.sync_copy(uidx_hbm.at[pl.ds(base + c * CB, CB)], uidx_v.at[c])
    pltpu.sync_copy(iidx_hbm.at[pl.ds(base + c * CB, CB)], iidx_v.at[c])
  copies = []
  for c in range(NCHUNK):
    copies.append(pltpu.async_copy(
        user_table.at[uidx_v.at[c]], rows_u.at[pl.ds(c * CB, CB)], sem))
    copies.append(pltpu.async_copy(
        item_table.at[iidx_v.at[c]], rows_v.at[pl.ds(c * CB, CB)], sem))
  for cp in copies:
    cp.wait()

  row_iota = lax.iota(jnp.int32, L)

  def group_body(g, _):
    row0 = pl.multiple_of(g * L, L)
    for k in range(L):
      r = row0 + k
      s = None
      for cc in range(D // L):
        u = rows_u[r, pl.ds(cc * L, L)]
        v = rows_v[r, pl.ds(cc * L, L)]
        m = u * v
        s = m if s is None else s + m
      pbuf[k, :] = s
    acc = jnp.zeros((L,), jnp.float32)
    for l in range(L):
      col = jnp.full((L,), l, jnp.int32)
      acc = acc + plsc.load_gather(pbuf, [row_iota, col])
    out_v[pl.ds(row0, L)] = acc
    return 0

  lax.fori_loop(0, BPW // L, group_body, 0)

  pltpu.sync_copy(out_v, out_hbm.at[pl.ds(base, BPW)])


@jax.jit
def _towers(user_indices, item_indices, user_table, item_table):
  mesh = plsc.VectorSubcoreMesh(core_axis_name="c", subcore_axis_name="s")
  f = pl.kernel(
      _tower_kernel,
      out_type=jax.ShapeDtypeStruct((B,), jnp.float32),
      mesh=mesh,
      compiler_params=pltpu.CompilerParams(
          needs_layout_passes=False, use_tc_tiling_on_sc=False),
      scratch_types=[
          pltpu.VMEM((NCHUNK, CB), jnp.int32),
          pltpu.VMEM((NCHUNK, CB), jnp.int32),
          pltpu.VMEM((BPW, D), jnp.float32),
          pltpu.VMEM((BPW, D), jnp.float32),
          pltpu.VMEM((L, L), jnp.float32),
          pltpu.VMEM((BPW,), jnp.float32),
          pltpu.SemaphoreType.DMA,
      ],
  )
  return f(user_table, item_table, user_indices, item_indices)


def kernel(user_indices, item_indices, user_table, item_table):
  return _towers(user_indices.astype(jnp.int32),
                 item_indices.astype(jnp.int32),
                 user_table, item_table)
